# baseline (speedup 1.0000x reference)
import functools

import jax
import jax.numpy as jnp
from jax import lax
from jax.experimental import pallas as pl
from jax.experimental.pallas import tpu as pltpu
from jax.experimental.pallas import tpu_sc as plsc
from jax.experimental.layout import Format, Layout, with_layout_constraint

_VOCAB = 100000
_NFIELDS = 26
_EMB = 64
_PAIR = 2 * _EMB
_B = 4096
_R = _B * _NFIELDS
_NW = 32
_RPW = _R // _NW
_CHUNK = 128
_WAVES = (7, 7, 7, 5)
_BUFROWS = max(_WAVES) * _CHUNK


def _sc_gather(tab_flat, xi_t):
    mesh = plsc.VectorSubcoreMesh(core_axis_name="c", subcore_axis_name="s")

    @functools.partial(
        pl.kernel,
        out_type=jax.ShapeDtypeStruct((_R, _EMB), jnp.float32),
        mesh=mesh,
        scratch_types=[
            pltpu.VMEM((_RPW,), jnp.int32),
            pltpu.VMEM((_BUFROWS, _EMB), jnp.float32),
            pltpu.VMEM((_BUFROWS, _EMB), jnp.float32),
            pltpu.SemaphoreType.DMA,
            pltpu.SemaphoreType.DMA,
            pltpu.SemaphoreType.DMA,
            pltpu.SemaphoreType.DMA,
        ],
        compiler_params=pltpu.CompilerParams(use_tc_tiling_on_sc=False),
    )
    def k(tab_hbm, xi_hbm, out_hbm,
          idx_v, buf0, buf1, gsem0, gsem1, osem0, osem1):
        wid = lax.axis_index("c") * 16 + lax.axis_index("s")
        base = wid * _RPW
        pltpu.sync_copy(xi_hbm.at[pl.ds(base, _RPW)], idx_v)

        def scale(i, carry):
            s = pl.ds(i * 16, 16)
            j = (base + i * 16) // _B
            idx_v[s] = lax.shift_left(idx_v[s] + j * _VOCAB, 1)
            return carry

        lax.fori_loop(0, _RPW // 16, scale, 0, unroll=4)

        woff = []
        acc = 0
        for nc in _WAVES:
            woff.append(acc)
            acc += nc

        bufs = (buf0, buf1)
        gsems = (gsem0, gsem1)
        osems = (osem0, osem1)

        def fire(wave):
            cps = []
            for c in range(_WAVES[wave]):
                row0 = (woff[wave] + c) * _CHUNK
                cp = pltpu.make_async_copy(
                    tab_hbm.at[idx_v.at[pl.ds(row0, _CHUNK)]],
                    bufs[wave % 2].at[pl.ds(c * _CHUNK, _CHUNK)],
                    gsems[wave % 2],
                )
                cp.start()
                cps.append(cp)
            return cps

        def out_copy(wave):
            rows = _WAVES[wave] * _CHUNK
            cp = pltpu.make_async_copy(
                bufs[wave % 2].at[pl.ds(0, rows)],
                out_hbm.at[pl.ds(base + woff[wave] * _CHUNK, rows)],
                osems[wave % 2])
            cp.start()
            return cp

        nwave = len(_WAVES)
        pending = [None, None]
        outs = [None, None]
        for w in range(nwave):
            b = w % 2
            if outs[b] is not None:
                outs[b].wait()
            pending[b] = fire(w)
            if w >= 1:
                pb = (w - 1) % 2
                for cp in pending[pb]:
                    cp.wait()
                outs[pb] = out_copy(w - 1)
        lb = (nwave - 1) % 2
        for cp in pending[lb]:
            cp.wait()
        outs[lb] = out_copy(nwave - 1)
        outs[0].wait()
        outs[1].wait()

    return k(tab_flat, xi_t)


def _mm(x, w):
    return lax.dot_general(x, w, (((1,), (1,)), ((), ())),
                           preferred_element_type=jnp.float32)


def _mlp_body(emb_ref, xv_ref, dw_ref, dw1_ref,
              bw1_ref, bb1_ref, bw2_ref, bb2_ref, bw3_ref, bb3_ref,
              cw1_ref, cb1_ref, cw2_ref, cb2_ref, cw3_ref, cb3_ref,
              te_ref, t0_ref, t1_ref, tb1_ref, tw2_ref, tb2_ref,
              tw3_ref, tb3_ref, out_ref):
    xv = xv_ref[...]
    relu = lambda v: jnp.maximum(v, 0.0)

    x0 = xv * dw_ref[...]
    h = relu(_mm(x0, bw1_ref[...]) + bb1_ref[...])
    h = relu(_mm(h, bw2_ref[...]) + bb2_ref[...])
    bot0 = relu(_mm(h, bw3_ref[...]) + bb3_ref[...])

    x1 = xv * dw1_ref[...]
    h = relu(_mm(x1, cw1_ref[...]) + cb1_ref[...])
    h = relu(_mm(h, cw2_ref[...]) + cb2_ref[...])
    bot1 = relu(_mm(h, cw3_ref[...]) + cb3_ref[...])

    t = _mm(bot0, t0_ref[...]) + _mm(bot1, t1_ref[...]) + tb1_ref[...]
    for p in range(_NFIELDS // 2):
        pair = jnp.concatenate([emb_ref[2 * p], emb_ref[2 * p + 1]], axis=1)
        t += _mm(pair, te_ref[:, pl.ds(p * _PAIR, _PAIR)])
    t = relu(t)
    t = relu(_mm(t, tw2_ref[...]) + tb2_ref[...])
    out = jnp.sum(t * tw3_ref[...], axis=1, keepdims=True) + tb3_ref[0, 0]
    out_ref[...] = out


def _tc_mlp(emb, xv_p, dw_p, dw1_p, bot, bot1, top):
    bm = 1024
    grid = (_B // bm,)
    full = lambda shape: pl.BlockSpec(shape, lambda i: tuple(0 for _ in shape))
    wspecs = []
    wargs = []
    for w in (*bot, *bot1, *top):
        wspecs.append(full(w.shape))
        wargs.append(w)
    return pl.pallas_call(
        _mlp_body,
        grid=grid,
        in_specs=[
            pl.BlockSpec((_NFIELDS, bm, _EMB), lambda i: (0, i, 0)),
            pl.BlockSpec((bm, 128), lambda i: (i, 0)),
            full(dw_p.shape),
            full(dw1_p.shape),
            *wspecs,
        ],
        out_specs=pl.BlockSpec((bm, 1), lambda i: (i, 0)),
        out_shape=jax.ShapeDtypeStruct((_B, 1), jnp.float32),
    )(emb, xv_p, dw_p, dw1_p, *wargs)


def kernel(Xi, Xv, emb_tables, dense_weight, dense_weight_1,
           bot_params, bot1_params, top_params):
    tab3 = with_layout_constraint(
        emb_tables,
        Layout(major_to_minor=(0, 1, 2), tiling=((8,),)))
    tab_flat = with_layout_constraint(
        tab3.reshape(_NFIELDS * _VOCAB, _EMB),
        Layout(major_to_minor=(0, 1), tiling=((8,),)))
    xi_t = Xi.reshape(_B, _NFIELDS).T.reshape(_R).astype(jnp.int32)

    emb = _sc_gather(tab_flat, xi_t).reshape(_NFIELDS, _B, _EMB)

    xv_p = jnp.pad(Xv, ((0, 0), (0, 128 - 13)))
    dw_p = jnp.pad(dense_weight, (0, 128 - 13)).reshape(1, 128)
    dw1_p = jnp.pad(dense_weight_1, (0, 128 - 13)).reshape(1, 128)

    def prep_mlp(params, pad_first_k=None):
        out = []
        n = len(params) // 2
        for i in range(n):
            w, b = params[2 * i], params[2 * i + 1]
            if i == 0 and pad_first_k is not None:
                w = jnp.pad(w, ((0, 0), (0, pad_first_k - w.shape[1])))
            out.append(w)
            out.append(b.reshape(1, -1))
        return out

    bot = prep_mlp(bot_params, pad_first_k=128)
    bot1 = prep_mlp(bot1_params, pad_first_k=128)

    tw1, tb1, tw2, tb2, tw3, tb3 = top_params
    ne = _NFIELDS * _EMB
    top = [
        tw1[:, :ne],
        tw1[:, ne:ne + _EMB],
        tw1[:, ne + _EMB:],
        tb1.reshape(1, -1),
        tw2, tb2.reshape(1, -1),
        tw3,
        tb3.reshape(1, 1),
    ]
    return _tc_mlp(emb, xv_p, dw_p, dw1_p, bot, bot1, top)


# speedup vs baseline: 1.0090x; 1.0090x over previous
import functools

import jax
import jax.numpy as jnp
from jax import lax
from jax.experimental import pallas as pl
from jax.experimental.pallas import tpu as pltpu
from jax.experimental.pallas import tpu_sc as plsc

_VOCAB = 100000
_NFIELDS = 26
_EMB = 64
_B = 4096
_R = _B * _NFIELDS
_NW = 32
_RPW = _R // _NW
_CHUNK = 104
_NCHUNK = 8
_WROWS = _CHUNK * _NCHUNK
_NWAVE = _RPW // _WROWS


def _sc_gather(flat_tables, flat_xi, offsets):
    mesh = plsc.VectorSubcoreMesh(core_axis_name="c", subcore_axis_name="s")

    @functools.partial(
        pl.kernel,
        out_type=jax.ShapeDtypeStruct((_R, _EMB), jnp.float32),
        mesh=mesh,
        scratch_types=[
            pltpu.VMEM((_RPW,), jnp.int32),
            pltpu.VMEM((_RPW,), jnp.int32),
            pltpu.VMEM((_WROWS, _EMB), jnp.float32),
            pltpu.VMEM((_WROWS, _EMB), jnp.float32),
            pltpu.SemaphoreType.DMA,
            pltpu.SemaphoreType.DMA,
            pltpu.SemaphoreType.DMA,
            pltpu.SemaphoreType.DMA,
        ],
        compiler_params=pltpu.CompilerParams(use_tc_tiling_on_sc=False),
    )
    def k(tab_hbm, xi_hbm, off_hbm, out_hbm,
          idx_v, off_v, buf0, buf1, gsem0, gsem1, osem0, osem1):
        wid = lax.axis_index("c") * 16 + lax.axis_index("s")
        base = wid * _RPW
        pltpu.sync_copy(xi_hbm.at[pl.ds(base, _RPW)], idx_v)
        pltpu.sync_copy(off_hbm, off_v)

        def add_off(i, carry):
            s = pl.ds(i * 16, 16)
            idx_v[s] = idx_v[s] + off_v[s]
            return carry

        lax.fori_loop(0, _RPW // 16, add_off, 0, unroll=4)

        def fire(buf, wave, sem):
            cps = []
            for c in range(_NCHUNK):
                row0 = wave * _WROWS + c * _CHUNK
                cp = pltpu.make_async_copy(
                    tab_hbm.at[idx_v.at[pl.ds(row0, _CHUNK)]],
                    buf.at[pl.ds(c * _CHUNK, _CHUNK)],
                    sem,
                )
                cp.start()
                cps.append(cp)
            return cps

        def out_copy(buf, wave, sem):
            cp = pltpu.make_async_copy(
                buf, out_hbm.at[pl.ds(base + wave * _WROWS, _WROWS)], sem)
            cp.start()
            return cp

        def drain(cps):
            for cp in cps:
                cp.wait()

        g0 = fire(buf0, 0, gsem0)
        g1 = fire(buf1, 1, gsem1)
        drain(g0)
        o0 = out_copy(buf0, 0, osem0)
        drain(g1)
        o1 = out_copy(buf1, 1, osem1)
        o0.wait()
        g2 = fire(buf0, 2, gsem0)
        o1.wait()
        g3 = fire(buf1, 3, gsem1)
        drain(g2)
        o2 = out_copy(buf0, 2, osem0)
        drain(g3)
        o3 = out_copy(buf1, 3, osem1)
        o2.wait()
        o3.wait()

    return k(flat_tables, flat_xi, offsets)


def _mm(x, w):
    return lax.dot_general(x, w, (((1,), (1,)), ((), ())),
                           preferred_element_type=jnp.float32)


def _mlp_body(emb_ref, xv_ref, dw_ref, dw1_ref,
              bw1_ref, bb1_ref, bw2_ref, bb2_ref, bw3_ref, bb3_ref,
              cw1_ref, cb1_ref, cw2_ref, cb2_ref, cw3_ref, cb3_ref,
              te_ref, t0_ref, t1_ref, tb1_ref, tw2_ref, tb2_ref,
              tw3_ref, tb3_ref, out_ref):
    xv = xv_ref[...]
    relu = lambda v: jnp.maximum(v, 0.0)

    x0 = xv * dw_ref[...]
    h = relu(_mm(x0, bw1_ref[...]) + bb1_ref[...])
    h = relu(_mm(h, bw2_ref[...]) + bb2_ref[...])
    bot0 = relu(_mm(h, bw3_ref[...]) + bb3_ref[...])

    x1 = xv * dw1_ref[...]
    h = relu(_mm(x1, cw1_ref[...]) + cb1_ref[...])
    h = relu(_mm(h, cw2_ref[...]) + cb2_ref[...])
    bot1 = relu(_mm(h, cw3_ref[...]) + cb3_ref[...])

    t = _mm(emb_ref[...], te_ref[...])
    t += _mm(bot0, t0_ref[...]) + _mm(bot1, t1_ref[...]) + tb1_ref[...]
    t = relu(t)
    t = relu(_mm(t, tw2_ref[...]) + tb2_ref[...])
    out = jnp.sum(t * tw3_ref[...], axis=1, keepdims=True) + tb3_ref[0, 0]
    out_ref[...] = out


def _tc_mlp(emb, xv_p, dw_p, dw1_p, bot, bot1, top):
    bm = 1024
    grid = (_B // bm,)
    full = lambda shape: pl.BlockSpec(shape, lambda i: tuple(0 for _ in shape))
    wspecs = []
    wargs = []
    for w in (*bot, *bot1, *top):
        wspecs.append(full(w.shape))
        wargs.append(w)
    return pl.pallas_call(
        _mlp_body,
        grid=grid,
        in_specs=[
            pl.BlockSpec((bm, _NFIELDS * _EMB), lambda i: (i, 0)),
            pl.BlockSpec((bm, 128), lambda i: (i, 0)),
            full(dw_p.shape),
            full(dw1_p.shape),
            *wspecs,
        ],
        out_specs=pl.BlockSpec((bm, 1), lambda i: (i, 0)),
        out_shape=jax.ShapeDtypeStruct((_B, 1), jnp.float32),
    )(emb, xv_p, dw_p, dw1_p, *wargs)


def kernel(Xi, Xv, emb_tables, dense_weight, dense_weight_1,
           bot_params, bot1_params, top_params):
    flat_tables = emb_tables.reshape(_NFIELDS * _VOCAB, _EMB)
    flat_xi = Xi.reshape(_R).astype(jnp.int32)
    offsets = jnp.tile(jnp.arange(_NFIELDS, dtype=jnp.int32) * _VOCAB,
                       _RPW // _NFIELDS)

    emb = _sc_gather(flat_tables, flat_xi, offsets).reshape(_B, _NFIELDS * _EMB)

    xv_p = jnp.pad(Xv, ((0, 0), (0, 128 - 13)))
    dw_p = jnp.pad(dense_weight, (0, 128 - 13)).reshape(1, 128)
    dw1_p = jnp.pad(dense_weight_1, (0, 128 - 13)).reshape(1, 128)

    def prep_mlp(params, pad_first_k=None):
        out = []
        n = len(params) // 2
        for i in range(n):
            w, b = params[2 * i], params[2 * i + 1]
            if i == 0 and pad_first_k is not None:
                w = jnp.pad(w, ((0, 0), (0, pad_first_k - w.shape[1])))
            out.append(w)
            out.append(b.reshape(1, -1))
        return out

    bot = prep_mlp(bot_params, pad_first_k=128)
    bot1 = prep_mlp(bot1_params, pad_first_k=128)

    tw1, tb1, tw2, tb2, tw3, tb3 = top_params
    ne = _NFIELDS * _EMB
    top = [
        tw1[:, :ne],
        tw1[:, ne:ne + _EMB],
        tw1[:, ne + _EMB:],
        tb1.reshape(1, -1),
        tw2, tb2.reshape(1, -1),
        tw3,
        tb3.reshape(1, 1),
    ]
    return _tc_mlp(emb, xv_p, dw_p, dw1_p, bot, bot1, top)
